# baseline (speedup 1.0000x reference)
import functools

import jax
import jax.numpy as jnp
from jax import lax
from jax.experimental import pallas as pl
from jax.experimental.pallas import tpu as pltpu
from jax.experimental.pallas import tpu_sc as plsc

_SEQ = 2048
_HEADS = 16
_NUM_BUCKETS = 32
_W = 2 * _SEQ
_BLK_I = 2048
_EROWS = 128
_TILE = 128
_LANES = 16
_NWORKERS = 32
_PER_W = _HEADS * _W // _NWORKERS


def _bucket_of(q):
    n = _SEQ - q
    ret = jnp.where(n < 0, _NUM_BUCKETS // 2, 0)
    na = jnp.abs(n)
    nsq = (na * na).astype(jnp.float32)
    e = (lax.bitcast_convert_type(nsq, jnp.int32) >> 23) - 127
    val_large = jnp.minimum(8 + (e - 6), 15)
    return ret + jnp.where(na < 8, na, val_large)


def _sc_w_body(table_hbm, w_hbm, tbl_v, buf_v):
    wid = lax.axis_index("s") * 2 + lax.axis_index("c")
    head = wid >> 1
    qbase = (wid & 1) * _PER_W
    pltpu.sync_copy(table_hbm, tbl_v)
    lanes = lax.iota(jnp.int32, _LANES)
    base = head * _NUM_BUCKETS
    t_lo = tbl_v[pl.ds(base, _LANES)]
    t_hi = tbl_v[pl.ds(base + _LANES, _LANES)]

    def step(i, carry):
        q = qbase + i * _LANES + lanes
        b = _bucket_of(q)
        bl = b & (_LANES - 1)
        vals = jnp.where(
            b < _LANES,
            t_lo.at[bl].get(mode="promise_in_bounds"),
            t_hi.at[bl].get(mode="promise_in_bounds"),
        )
        buf_v[pl.ds(i * _LANES, _LANES)] = vals
        return carry

    lax.fori_loop(0, _PER_W // _LANES, step, 0)
    pltpu.sync_copy(buf_v, w_hbm.at[pl.ds(wid * _PER_W, _PER_W)])


@functools.cache
def _make_sc_w():
    return pl.kernel(
        _sc_w_body,
        mesh=plsc.VectorSubcoreMesh(core_axis_name="c", subcore_axis_name="s"),
        out_type=jax.ShapeDtypeStruct((_HEADS * _W,), jnp.float32),
        scratch_types=[
            pltpu.VMEM((_NUM_BUCKETS * _HEADS,), jnp.float32),
            pltpu.VMEM((_PER_W,), jnp.float32),
        ],
    )


def _tc_expand_body(w_ref, out_ref, e_ref):
    wv = w_ref[0, :, :]
    wb = jnp.broadcast_to(wv, (_EROWS, _W))
    e_ref[:, :] = pltpu.roll(wb, 0, axis=1, stride=1, stride_axis=0)
    nt = _SEQ // _TILE
    for gib in range(_BLK_I // _TILE):
        r0 = _TILE * gib
        start = pl.multiple_of(_TILE * nt - _TILE * gib, _TILE)
        out_ref[0, 0, r0:r0 + _TILE, :] = e_ref[:, pl.ds(start, _SEQ)]


def kernel(qk_dots, table):
    del qk_dots
    w = _make_sc_w()(table.T.reshape(-1))
    w3 = w.reshape(_HEADS, 1, _W)
    out = pl.pallas_call(
        _tc_expand_body,
        grid=(_HEADS,),
        in_specs=[pl.BlockSpec((1, 1, _W), lambda h: (h, 0, 0))],
        out_specs=pl.BlockSpec((1, 1, _BLK_I, _SEQ), lambda h: (0, h, 0, 0)),
        out_shape=jax.ShapeDtypeStruct((1, _HEADS, _SEQ, _SEQ), jnp.float32),
        scratch_shapes=[pltpu.VMEM((_EROWS, _W), jnp.float32)],
    )(w3)
    return out


# speedup vs baseline: 1.2621x; 1.2621x over previous
import jax
import jax.numpy as jnp
from jax.experimental import pallas as pl
from jax.experimental.pallas import tpu as pltpu

_SEQ = 2048
_HEADS = 16
_NUM_BUCKETS = 32
_W = 2 * _SEQ
_EROWS = 128
_TILE = 128
_NB = _SEQ // _TILE


def _band_copy(e_ref, out_ref, h, gib, sem):
    start = pl.multiple_of(_TILE * _NB - _TILE * gib, _TILE)
    return pltpu.make_async_copy(
        e_ref.at[:, pl.ds(start, _SEQ)],
        out_ref.at[0, h, pl.ds(gib * _TILE, _TILE), :],
        sem,
    )


def _bias_body(tbl_ref, out_ref, e0_ref, e1_ref, sem_ref):
    step = pl.program_id(0)
    h = jnp.minimum(step, _HEADS - 1)

    @pl.when(step >= 2)
    def _drain():
        hprev = step - 2
        for gib in range(_NB):
            @pl.when(hprev % 2 == 0)
            def _():
                _band_copy(e0_ref, out_ref, hprev, gib, sem_ref.at[0]).wait()

            @pl.when(hprev % 2 == 1)
            def _():
                _band_copy(e1_ref, out_ref, hprev, gib, sem_ref.at[1]).wait()

    @pl.when(step < _HEADS)
    def _build_and_issue():
        z = jax.lax.broadcasted_iota(jnp.int32, (1, _W), 1)
        n = _SEQ - z
        half = _NUM_BUCKETS // 2
        ret = jnp.where(n < 0, half, 0)
        na = jnp.abs(n)
        max_exact = half // 2
        nsq = (na * na).astype(jnp.float32)
        e = (jax.lax.bitcast_convert_type(nsq, jnp.int32) >> 23) - 127
        val_large = jnp.minimum(max_exact + (e - 6), half - 1)
        bucket = ret + jnp.where(na < max_exact, na, val_large)
        w = jnp.zeros((1, _W), jnp.float32)
        for b in range(_NUM_BUCKETS):
            w = w + jnp.where(bucket == b, tbl_ref[0, 0, b], 0.0)
        wb = jnp.broadcast_to(w, (_EROWS, _W))
        rolled = pltpu.roll(wb, 0, axis=1, stride=1, stride_axis=0)

        @pl.when(h % 2 == 0)
        def _():
            e0_ref[:, :] = rolled
            for gib in range(_NB):
                _band_copy(e0_ref, out_ref, h, gib, sem_ref.at[0]).start()

        @pl.when(h % 2 == 1)
        def _():
            e1_ref[:, :] = rolled
            for gib in range(_NB):
                _band_copy(e1_ref, out_ref, h, gib, sem_ref.at[1]).start()


def kernel(qk_dots, table):
    del qk_dots
    tbl_t = table.T.reshape(_HEADS, 1, _NUM_BUCKETS)
    out = pl.pallas_call(
        _bias_body,
        grid=(_HEADS + 2,),
        in_specs=[
            pl.BlockSpec((1, 1, _NUM_BUCKETS),
                         lambda s: (jnp.minimum(s, _HEADS - 1), 0, 0),
                         memory_space=pltpu.SMEM),
        ],
        out_specs=pl.BlockSpec(memory_space=pl.ANY),
        out_shape=jax.ShapeDtypeStruct((1, _HEADS, _SEQ, _SEQ), jnp.float32),
        scratch_shapes=[
            pltpu.VMEM((_EROWS, _W), jnp.float32),
            pltpu.VMEM((_EROWS, _W), jnp.float32),
            pltpu.SemaphoreType.DMA((2,)),
        ],
    )(tbl_t)
    return out
